# baseline (speedup 1.0000x reference)
import functools

import jax
import jax.numpy as jnp
from jax import lax
from jax.experimental import pallas as pl
from jax.experimental.pallas import tpu as pltpu
from jax.experimental.pallas import tpu_sc as plsc

N_ANCHORS = 5000
N_PAD = 5120
ROWS, COLS = 40, 128
K_MAX = 100
IMG_H = IMG_W = 1024.0
SCORE_THRESH = 0.5
IOU_THRESH = 0.1
NEG = -1.0e30

NTILES = 16
PER_TILE = N_PAD // NTILES
VPER = PER_TILE // 16
SLOT = 128


def _decode_body(ax1, ay1, ax2, ay2, r0, r1, r2, r3, sc, out_ref):
    ax1 = ax1[...]
    ay1 = ay1[...]
    ax2 = ax2[...]
    ay2 = ay2[...]
    widths = ax2 - ax1
    heights = ay2 - ay1
    ctr_x = ax1 + 0.5 * widths
    ctr_y = ay1 + 0.5 * heights
    dx = r0[...] * 0.1
    dy = r1[...] * 0.1
    dw = r2[...] * 0.2
    dh = r3[...] * 0.2
    pred_ctr_x = ctr_x + dx * widths
    pred_ctr_y = ctr_y + dy * heights
    pred_w = jnp.exp(dw) * widths
    pred_h = jnp.exp(dh) * heights
    x1 = jnp.clip(pred_ctr_x - 0.5 * pred_w, 0.0, IMG_W)
    y1 = jnp.clip(pred_ctr_y - 0.5 * pred_h, 0.0, IMG_H)
    x2 = jnp.clip(pred_ctr_x + 0.5 * pred_w, 0.0, IMG_W)
    y2 = jnp.clip(pred_ctr_y + 0.5 * pred_h, 0.0, IMG_H)
    areas = jnp.maximum(x2 - x1, 0.0) * jnp.maximum(y2 - y1, 0.0)
    scores = sc[...]
    masked = jnp.where(scores > SCORE_THRESH, scores, NEG)
    out_ref[0:ROWS, :] = x1
    out_ref[ROWS:2 * ROWS, :] = y1
    out_ref[2 * ROWS:3 * ROWS, :] = x2
    out_ref[3 * ROWS:4 * ROWS, :] = y2
    out_ref[4 * ROWS:5 * ROWS, :] = areas
    out_ref[5 * ROWS:6 * ROWS, :] = masked


def _sc_nms(data_hbm, out_hbm, bufv, cbuf, stage, shared, rb, outv):
    cid = lax.axis_index("c")
    tid = lax.axis_index("s")
    lane = lax.iota(jnp.int32, 16)

    @pl.when(cid == 0)
    def _():
        base = tid * PER_TILE
        for f in range(6):
            pltpu.sync_copy(data_hbm.at[pl.ds(f * N_PAD + base, PER_TILE)],
                            bufv.at[pl.ds(f * PER_TILE, PER_TILE)])

        zero16 = jnp.zeros((16,), jnp.float32)
        neg16 = jnp.full((16,), NEG, jnp.float32)
        for q in range(VPER):
            cbuf[pl.ds(6 * PER_TILE + q * 16, 16)] = neg16
        for q in range(VPER):
            cbuf[pl.ds(4 * PER_TILE + q * 16, 16)] = zero16
        for q in range(SLOT // 16):
            stage[pl.ds(q * 16, 16)] = zero16

        @pl.when(tid == 0)
        def _():
            for r in range(K_MAX):
                outv[pl.ds(r * 16, 16)] = zero16

        off = jnp.int32(0)
        for j in range(VPER):
            v = bufv[pl.ds(5 * PER_TILE + j * 16, 16)]
            mask = v > NEG * 0.5
            for f in range(5):
                plsc.store_compressed(
                    cbuf.at[pl.ds(f * PER_TILE + off, 16)],
                    bufv[pl.ds(f * PER_TILE + j * 16, 16)], mask=mask)
            gidx = (lane + (base + j * 16)).astype(jnp.float32)
            plsc.store_compressed(
                cbuf.at[pl.ds(5 * PER_TILE + off, 16)], gidx, mask=mask)
            plsc.store_compressed(
                cbuf.at[pl.ds(6 * PER_TILE + off, 16)], v, mask=mask)
            off = off + plsc.all_reduce_population_count(mask)[0]
        nv = lax.shift_right_logical(off + 15, 4)

        fmap = (jnp.minimum(lane, 4) * PER_TILE
                + jnp.where(lane == 5, 6 * PER_TILE - 4 * PER_TILE, 0)
                + jnp.where(lane == 6, 5 * PER_TILE - 4 * PER_TILE, 0))

        def local_rec(bestv, besti):
            m_loc = jnp.max(bestv)
            i_loc = jnp.min(jnp.where(bestv == m_loc, besti, jnp.int32(1 << 30)))
            g = plsc.load_gather(cbuf, [fmap + i_loc])
            return jnp.where(lane >= 7, zero16, g)

        def init_scan(j, st):
            bestv, besti = st
            v = cbuf[pl.ds(6 * PER_TILE + j * 16, 16)]
            upd = v > bestv
            besti = jnp.where(upd, lane + j * 16, besti)
            bestv = jnp.where(upd, v, bestv)
            return bestv, besti

        bestv0, besti0 = lax.fori_loop(
            0, nv, init_scan,
            (jnp.full((16,), NEG, jnp.float32), jnp.zeros((16,), jnp.int32)))
        rec0 = local_rec(bestv0, besti0)

        def body(k, carry):
            nk, rec = carry
            buf = lax.rem(k, 2)
            stage[pl.ds(0, 16)] = rec
            pltpu.sync_copy(stage, shared.at[buf, tid])
            plsc.subcore_barrier()
            pltpu.sync_copy(shared.at[buf], rb)
            five = jnp.full((16,), 5, jnp.int32)
            sc16 = plsc.load_gather(rb, [lane, five])
            ix16 = plsc.load_gather(rb, [lane, five + 1])
            m = jnp.max(sc16)
            has = m > NEG * 0.5
            wix = jnp.min(jnp.where(sc16 == m, ix16, jnp.float32(1e9)))
            onehot = (sc16 == m) & (ix16 == wix)
            tsel = jnp.min(jnp.where(onehot, lane, jnp.int32(99)))
            tsplat = jnp.full((16,), tsel, jnp.int32)
            wrec = plsc.load_gather(rb, [tsplat, jnp.minimum(lane, 4)])
            bx1 = plsc.load_gather(rb, [tsplat, jnp.zeros((16,), jnp.int32)])
            by1 = plsc.load_gather(rb, [tsplat, jnp.full((16,), 1, jnp.int32)])
            bx2 = plsc.load_gather(rb, [tsplat, jnp.full((16,), 2, jnp.int32)])
            by2 = plsc.load_gather(rb, [tsplat, jnp.full((16,), 3, jnp.int32)])
            bar = plsc.load_gather(rb, [tsplat, jnp.full((16,), 4, jnp.int32)])
            hasv = jnp.full((16,), jnp.where(has, 1.0, 0.0)) > 0.5

            def sup_scan(j, st):
                bestv, besti = st
                xx1 = jnp.maximum(bx1, cbuf[pl.ds(j * 16, 16)])
                yy1 = jnp.maximum(by1, cbuf[pl.ds(PER_TILE + j * 16, 16)])
                xx2 = jnp.minimum(bx2, cbuf[pl.ds(2 * PER_TILE + j * 16, 16)])
                yy2 = jnp.minimum(by2, cbuf[pl.ds(3 * PER_TILE + j * 16, 16)])
                av = cbuf[pl.ds(4 * PER_TILE + j * 16, 16)]
                inter = jnp.maximum(xx2 - xx1, 0.0) * jnp.maximum(yy2 - yy1, 0.0)
                iou = inter / jnp.maximum(bar + av - inter, 1e-9)
                supp = (iou > IOU_THRESH) & hasv
                v = jnp.where(supp, NEG, cbuf[pl.ds(6 * PER_TILE + j * 16, 16)])
                cbuf[pl.ds(6 * PER_TILE + j * 16, 16)] = v
                upd = v > bestv
                besti = jnp.where(upd, lane + j * 16, besti)
                bestv = jnp.where(upd, v, bestv)
                return bestv, besti

            bestv, besti = lax.fori_loop(
                0, nv, sup_scan,
                (jnp.full((16,), NEG, jnp.float32),
                 jnp.zeros((16,), jnp.int32)))
            rec_next = local_rec(bestv, besti)

            @pl.when((tid == 0) & has)
            def _():
                row = jnp.where(lane < 4, wrec,
                      jnp.where(lane == 4, jnp.full((16,), m),
                                jnp.zeros((16,), jnp.float32)))
                outv[pl.ds(nk * 16, 16)] = row

            nk = nk + jnp.where(has, 1, 0).astype(jnp.int32)
            return nk, rec_next

        lax.fori_loop(0, K_MAX, body, (jnp.int32(0), rec0))

        @pl.when(tid == 0)
        def _():
            pltpu.sync_copy(outv, out_hbm)


_sc_mesh = plsc.VectorSubcoreMesh(
    core_axis_name="c", subcore_axis_name="s", num_cores=2, num_subcores=16)

_sc_call = pl.kernel(
    _sc_nms,
    out_type=jax.ShapeDtypeStruct((K_MAX * 16,), jnp.float32),
    mesh=_sc_mesh,
    compiler_params=pltpu.CompilerParams(needs_layout_passes=False),
    scratch_types=[
        pltpu.VMEM((6 * PER_TILE,), jnp.float32),
        pltpu.VMEM((7 * PER_TILE,), jnp.float32),
        pltpu.VMEM((SLOT,), jnp.float32),
        pltpu.VMEM_SHARED((2, NTILES, SLOT), jnp.float32),
        pltpu.VMEM((NTILES, SLOT), jnp.float32),
        pltpu.VMEM((K_MAX * 16,), jnp.float32),
    ],
)


def _pad2d(v):
    return jnp.pad(v, (0, N_PAD - N_ANCHORS)).reshape(ROWS, COLS)


@jax.jit
def kernel(classification, regression, anchors):
    a = anchors[0]
    r = regression[0]
    parts = [_pad2d(a[:, k]) for k in range(4)]
    parts += [_pad2d(r[:, k]) for k in range(4)]
    parts.append(_pad2d(classification[0, :, 1]))
    planes = pl.pallas_call(
        _decode_body,
        out_shape=jax.ShapeDtypeStruct((6 * ROWS, COLS), jnp.float32),
        in_specs=[pl.BlockSpec((ROWS, COLS), lambda: (0, 0))] * 9,
        out_specs=pl.BlockSpec((6 * ROWS, COLS), lambda: (0, 0)),
    )(*parts)
    out = _sc_call(planes.reshape(6 * N_PAD))
    return out.reshape(K_MAX, 16)[:, :5]


# speedup vs baseline: 1.0745x; 1.0745x over previous
import functools

import jax
import jax.numpy as jnp
from jax import lax
from jax.experimental import pallas as pl
from jax.experimental.pallas import tpu as pltpu
from jax.experimental.pallas import tpu_sc as plsc

N_ANCHORS = 5000
N_PAD = 5120
ROWS, COLS = 40, 128
K_MAX = 100
IMG_H = IMG_W = 1024.0
SCORE_THRESH = 0.5
IOU_THRESH = 0.1
NEG = -1.0e30

NTILES = 16
PER_TILE = N_PAD // NTILES
VPER = PER_TILE // 16
SLOT = 128


def _decode_body(ax1, ay1, ax2, ay2, r0, r1, r2, r3, sc, out_ref):
    ax1 = ax1[...]
    ay1 = ay1[...]
    ax2 = ax2[...]
    ay2 = ay2[...]
    widths = ax2 - ax1
    heights = ay2 - ay1
    ctr_x = ax1 + 0.5 * widths
    ctr_y = ay1 + 0.5 * heights
    dx = r0[...] * 0.1
    dy = r1[...] * 0.1
    dw = r2[...] * 0.2
    dh = r3[...] * 0.2
    pred_ctr_x = ctr_x + dx * widths
    pred_ctr_y = ctr_y + dy * heights
    pred_w = jnp.exp(dw) * widths
    pred_h = jnp.exp(dh) * heights
    x1 = jnp.clip(pred_ctr_x - 0.5 * pred_w, 0.0, IMG_W)
    y1 = jnp.clip(pred_ctr_y - 0.5 * pred_h, 0.0, IMG_H)
    x2 = jnp.clip(pred_ctr_x + 0.5 * pred_w, 0.0, IMG_W)
    y2 = jnp.clip(pred_ctr_y + 0.5 * pred_h, 0.0, IMG_H)
    areas = jnp.maximum(x2 - x1, 0.0) * jnp.maximum(y2 - y1, 0.0)
    scores = sc[...]
    masked = jnp.where(scores > SCORE_THRESH, scores, NEG)
    out_ref[0:ROWS, :] = x1
    out_ref[ROWS:2 * ROWS, :] = y1
    out_ref[2 * ROWS:3 * ROWS, :] = x2
    out_ref[3 * ROWS:4 * ROWS, :] = y2
    out_ref[4 * ROWS:5 * ROWS, :] = areas
    out_ref[5 * ROWS:6 * ROWS, :] = masked


def _sc_nms(data_hbm, out_hbm, bufv, cbuf, stage, shared, rb, outv):
    cid = lax.axis_index("c")
    tid = lax.axis_index("s")
    lane = lax.iota(jnp.int32, 16)

    @pl.when(cid == 0)
    def _():
        base = tid * PER_TILE
        for f in range(6):
            pltpu.sync_copy(data_hbm.at[pl.ds(f * N_PAD + base, PER_TILE)],
                            bufv.at[pl.ds(f * PER_TILE, PER_TILE)])

        zero16 = jnp.zeros((16,), jnp.float32)
        neg16 = jnp.full((16,), NEG, jnp.float32)
        for q in range(VPER):
            cbuf[pl.ds(6 * PER_TILE + q * 16, 16)] = neg16
        for q in range(VPER):
            cbuf[pl.ds(4 * PER_TILE + q * 16, 16)] = zero16
        for q in range(SLOT // 16):
            stage[pl.ds(q * 16, 16)] = zero16

        @pl.when(tid == 0)
        def _():
            for r in range(K_MAX):
                outv[pl.ds(r * 16, 16)] = zero16

        off = jnp.int32(0)
        for j in range(VPER):
            v = bufv[pl.ds(5 * PER_TILE + j * 16, 16)]
            mask = v > NEG * 0.5
            for f in range(5):
                plsc.store_compressed(
                    cbuf.at[pl.ds(f * PER_TILE + off, 16)],
                    bufv[pl.ds(f * PER_TILE + j * 16, 16)], mask=mask)
            gidx = (lane + (base + j * 16)).astype(jnp.float32)
            plsc.store_compressed(
                cbuf.at[pl.ds(5 * PER_TILE + off, 16)], gidx, mask=mask)
            plsc.store_compressed(
                cbuf.at[pl.ds(6 * PER_TILE + off, 16)], v, mask=mask)
            off = off + plsc.all_reduce_population_count(mask)[0]
        nv4 = lax.shift_right_logical(off + 63, 6)

        fmap = (jnp.minimum(lane, 4) * PER_TILE
                + jnp.where(lane == 5, 6 * PER_TILE - 4 * PER_TILE, 0)
                + jnp.where(lane == 6, 5 * PER_TILE - 4 * PER_TILE, 0))

        def local_rec(bestv, besti):
            m_loc = jnp.max(bestv)
            i_loc = jnp.min(jnp.where(bestv == m_loc, besti, jnp.int32(1 << 30)))
            g = plsc.load_gather(cbuf, [fmap + i_loc])
            return jnp.where(lane >= 7, zero16, g)

        def init_scan(j, st):
            bestv, besti = st
            for u in range(4):
                v = cbuf[pl.ds(6 * PER_TILE + j * 64 + u * 16, 16)]
                upd = v > bestv
                besti = jnp.where(upd, lane + j * 64 + u * 16, besti)
                bestv = jnp.where(upd, v, bestv)
            return bestv, besti

        bestv0, besti0 = lax.fori_loop(
            0, nv4, init_scan,
            (jnp.full((16,), NEG, jnp.float32), jnp.zeros((16,), jnp.int32)))
        rec0 = local_rec(bestv0, besti0)

        def body(k, carry):
            nk, rec = carry
            buf = lax.rem(k, 2)
            stage[pl.ds(0, 16)] = rec
            pltpu.sync_copy(stage, shared.at[buf, tid])
            plsc.subcore_barrier()
            pltpu.sync_copy(shared.at[buf], rb)
            five = jnp.full((16,), 5, jnp.int32)
            sc16 = plsc.load_gather(rb, [lane, five])
            ix16 = plsc.load_gather(rb, [lane, five + 1])
            m = jnp.max(sc16)
            has = m > NEG * 0.5
            wix = jnp.min(jnp.where(sc16 == m, ix16, jnp.float32(1e9)))
            onehot = (sc16 == m) & (ix16 == wix)
            tsel = jnp.min(jnp.where(onehot, lane, jnp.int32(99)))
            tsplat = jnp.full((16,), tsel, jnp.int32)
            wrec = plsc.load_gather(rb, [tsplat, jnp.minimum(lane, 4)])
            bx1 = plsc.load_gather(rb, [tsplat, jnp.zeros((16,), jnp.int32)])
            by1 = plsc.load_gather(rb, [tsplat, jnp.full((16,), 1, jnp.int32)])
            bx2 = plsc.load_gather(rb, [tsplat, jnp.full((16,), 2, jnp.int32)])
            by2 = plsc.load_gather(rb, [tsplat, jnp.full((16,), 3, jnp.int32)])
            bar = plsc.load_gather(rb, [tsplat, jnp.full((16,), 4, jnp.int32)])
            hasv = jnp.full((16,), jnp.where(has, 1.0, 0.0)) > 0.5

            def sup_scan(j, st):
                bestv, besti = st
                for u in range(4):
                    o = j * 64 + u * 16
                    xx1 = jnp.maximum(bx1, cbuf[pl.ds(o, 16)])
                    yy1 = jnp.maximum(by1, cbuf[pl.ds(PER_TILE + o, 16)])
                    xx2 = jnp.minimum(bx2, cbuf[pl.ds(2 * PER_TILE + o, 16)])
                    yy2 = jnp.minimum(by2, cbuf[pl.ds(3 * PER_TILE + o, 16)])
                    av = cbuf[pl.ds(4 * PER_TILE + o, 16)]
                    inter = jnp.maximum(xx2 - xx1, 0.0) * jnp.maximum(yy2 - yy1, 0.0)
                    iou = inter / jnp.maximum(bar + av - inter, 1e-9)
                    supp = (iou > IOU_THRESH) & hasv
                    v = jnp.where(supp, NEG, cbuf[pl.ds(6 * PER_TILE + o, 16)])
                    cbuf[pl.ds(6 * PER_TILE + o, 16)] = v
                    upd = v > bestv
                    besti = jnp.where(upd, lane + o, besti)
                    bestv = jnp.where(upd, v, bestv)
                return bestv, besti

            bestv, besti = lax.fori_loop(
                0, nv4, sup_scan,
                (jnp.full((16,), NEG, jnp.float32),
                 jnp.zeros((16,), jnp.int32)))
            rec_next = local_rec(bestv, besti)

            @pl.when((tid == 0) & has)
            def _():
                row = jnp.where(lane < 4, wrec,
                      jnp.where(lane == 4, jnp.full((16,), m),
                                jnp.zeros((16,), jnp.float32)))
                outv[pl.ds(nk * 16, 16)] = row

            nk = nk + jnp.where(has, 1, 0).astype(jnp.int32)
            return nk, rec_next

        lax.fori_loop(0, K_MAX, body, (jnp.int32(0), rec0))

        @pl.when(tid == 0)
        def _():
            pltpu.sync_copy(outv, out_hbm)


_sc_mesh = plsc.VectorSubcoreMesh(
    core_axis_name="c", subcore_axis_name="s", num_cores=2, num_subcores=16)

_sc_call = pl.kernel(
    _sc_nms,
    out_type=jax.ShapeDtypeStruct((K_MAX * 16,), jnp.float32),
    mesh=_sc_mesh,
    compiler_params=pltpu.CompilerParams(needs_layout_passes=False),
    scratch_types=[
        pltpu.VMEM((6 * PER_TILE,), jnp.float32),
        pltpu.VMEM((7 * PER_TILE,), jnp.float32),
        pltpu.VMEM((SLOT,), jnp.float32),
        pltpu.VMEM_SHARED((2, NTILES, SLOT), jnp.float32),
        pltpu.VMEM((NTILES, SLOT), jnp.float32),
        pltpu.VMEM((K_MAX * 16,), jnp.float32),
    ],
)


def _pad2d(v):
    return jnp.pad(v, (0, N_PAD - N_ANCHORS)).reshape(ROWS, COLS)


@jax.jit
def kernel(classification, regression, anchors):
    a = anchors[0]
    r = regression[0]
    parts = [_pad2d(a[:, k]) for k in range(4)]
    parts += [_pad2d(r[:, k]) for k in range(4)]
    parts.append(_pad2d(classification[0, :, 1]))
    planes = pl.pallas_call(
        _decode_body,
        out_shape=jax.ShapeDtypeStruct((6 * ROWS, COLS), jnp.float32),
        in_specs=[pl.BlockSpec((ROWS, COLS), lambda: (0, 0))] * 9,
        out_specs=pl.BlockSpec((6 * ROWS, COLS), lambda: (0, 0)),
    )(*parts)
    out = _sc_call(planes.reshape(6 * N_PAD))
    return out.reshape(K_MAX, 16)[:, :5]


# speedup vs baseline: 1.1817x; 1.0998x over previous
import functools

import jax
import jax.numpy as jnp
from jax import lax
from jax.experimental import pallas as pl
from jax.experimental.pallas import tpu as pltpu
from jax.experimental.pallas import tpu_sc as plsc

N_ANCHORS = 5000
N_PAD = 5120
ROWS, COLS = 40, 128
K_MAX = 100
IMG_H = IMG_W = 1024.0
SCORE_THRESH = 0.5
IOU_THRESH = 0.1
NEG = -1.0e30

NTILES = 16
PER_TILE = N_PAD // NTILES
VPER = PER_TILE // 16
SLOT = 128


def _decode_body(ax1, ay1, ax2, ay2, r0, r1, r2, r3, sc, out_ref):
    ax1 = ax1[...]
    ay1 = ay1[...]
    ax2 = ax2[...]
    ay2 = ay2[...]
    widths = ax2 - ax1
    heights = ay2 - ay1
    ctr_x = ax1 + 0.5 * widths
    ctr_y = ay1 + 0.5 * heights
    dx = r0[...] * 0.1
    dy = r1[...] * 0.1
    dw = r2[...] * 0.2
    dh = r3[...] * 0.2
    pred_ctr_x = ctr_x + dx * widths
    pred_ctr_y = ctr_y + dy * heights
    pred_w = jnp.exp(dw) * widths
    pred_h = jnp.exp(dh) * heights
    x1 = jnp.clip(pred_ctr_x - 0.5 * pred_w, 0.0, IMG_W)
    y1 = jnp.clip(pred_ctr_y - 0.5 * pred_h, 0.0, IMG_H)
    x2 = jnp.clip(pred_ctr_x + 0.5 * pred_w, 0.0, IMG_W)
    y2 = jnp.clip(pred_ctr_y + 0.5 * pred_h, 0.0, IMG_H)
    areas = jnp.maximum(x2 - x1, 0.0) * jnp.maximum(y2 - y1, 0.0)
    scores = sc[...]
    masked = jnp.where(scores > SCORE_THRESH, scores, NEG)
    out_ref[0:ROWS, :] = x1
    out_ref[ROWS:2 * ROWS, :] = y1
    out_ref[2 * ROWS:3 * ROWS, :] = x2
    out_ref[3 * ROWS:4 * ROWS, :] = y2
    out_ref[4 * ROWS:5 * ROWS, :] = areas
    out_ref[5 * ROWS:6 * ROWS, :] = masked


def _sc_nms(data_hbm, out_hbm, bufv, stage, shared, rb, outv):
    cid = lax.axis_index("c")
    tid = lax.axis_index("s")
    lane = lax.iota(jnp.int32, 16)

    @pl.when(cid == 0)
    def _():
        base = tid * PER_TILE
        for f in range(6):
            pltpu.sync_copy(data_hbm.at[pl.ds(f * N_PAD + base, PER_TILE)],
                            bufv.at[pl.ds(f * PER_TILE, PER_TILE)])

        zero16 = jnp.zeros((16,), jnp.float32)
        for q in range(SLOT // 16):
            stage[pl.ds(q * 16, 16)] = zero16

        @pl.when(tid == 0)
        def _():
            for r in range(K_MAX):
                outv[pl.ds(r * 16, 16)] = zero16

        def local_rec(bestv, besti):
            m_loc = jnp.max(bestv)
            i_loc = jnp.min(jnp.where(bestv == m_loc, besti, jnp.int32(1 << 30)))
            g = plsc.load_gather(
                bufv, [jnp.minimum(lane, 4) * PER_TILE + i_loc])
            gidx_f = (base + i_loc).astype(jnp.float32)
            return jnp.where(lane == 5, jnp.full((16,), m_loc),
                   jnp.where(lane == 6, jnp.full((16,), gidx_f), g))

        bestv = jnp.full((16,), NEG, jnp.float32)
        besti = jnp.zeros((16,), jnp.int32)
        sc_regs = []
        for j in range(VPER):
            v = bufv[pl.ds(5 * PER_TILE + j * 16, 16)]
            upd = v > bestv
            besti = jnp.where(upd, lane + (j * 16), besti)
            bestv = jnp.where(upd, v, bestv)
            sc_regs.append(v)
        rec0 = local_rec(bestv, besti)

        def body(k, carry):
            nk, rec = carry[0], carry[1]
            scs = carry[2:]
            buf = lax.rem(k, 2)
            stage[pl.ds(0, 16)] = rec
            pltpu.sync_copy(stage, shared.at[buf, tid])
            plsc.subcore_barrier()
            pltpu.sync_copy(shared.at[buf], rb)
            five = jnp.full((16,), 5, jnp.int32)
            sc16 = plsc.load_gather(rb, [lane, five])
            ix16 = plsc.load_gather(rb, [lane, five + 1])
            m = jnp.max(sc16)
            has = m > NEG * 0.5
            wix = jnp.min(jnp.where(sc16 == m, ix16, jnp.float32(1e9)))
            onehot = (sc16 == m) & (ix16 == wix)
            tsel = jnp.min(jnp.where(onehot, lane, jnp.int32(99)))
            tsplat = jnp.full((16,), tsel, jnp.int32)
            wrec = plsc.load_gather(rb, [tsplat, jnp.minimum(lane, 4)])
            bx1 = plsc.load_gather(rb, [tsplat, jnp.zeros((16,), jnp.int32)])
            by1 = plsc.load_gather(rb, [tsplat, jnp.full((16,), 1, jnp.int32)])
            bx2 = plsc.load_gather(rb, [tsplat, jnp.full((16,), 2, jnp.int32)])
            by2 = plsc.load_gather(rb, [tsplat, jnp.full((16,), 3, jnp.int32)])
            bar = plsc.load_gather(rb, [tsplat, jnp.full((16,), 4, jnp.int32)])
            hasv = jnp.full((16,), jnp.where(has, 1.0, 0.0)) > 0.5
            bestv = jnp.full((16,), NEG, jnp.float32)
            besti = jnp.zeros((16,), jnp.int32)
            new_scs = []
            for j in range(VPER):
                xx1 = jnp.maximum(bx1, bufv[pl.ds(j * 16, 16)])
                yy1 = jnp.maximum(by1, bufv[pl.ds(PER_TILE + j * 16, 16)])
                xx2 = jnp.minimum(bx2, bufv[pl.ds(2 * PER_TILE + j * 16, 16)])
                yy2 = jnp.minimum(by2, bufv[pl.ds(3 * PER_TILE + j * 16, 16)])
                av = bufv[pl.ds(4 * PER_TILE + j * 16, 16)]
                inter = jnp.maximum(xx2 - xx1, 0.0) * jnp.maximum(yy2 - yy1, 0.0)
                iou = inter / jnp.maximum(bar + av - inter, 1e-9)
                supp = (iou > IOU_THRESH) & hasv
                v = jnp.where(supp, NEG, scs[j])
                upd = v > bestv
                besti = jnp.where(upd, lane + (j * 16), besti)
                bestv = jnp.where(upd, v, bestv)
                new_scs.append(v)
            rec_next = local_rec(bestv, besti)

            @pl.when((tid == 0) & has)
            def _():
                row = jnp.where(lane < 4, wrec,
                      jnp.where(lane == 4, jnp.full((16,), m),
                                jnp.zeros((16,), jnp.float32)))
                outv[pl.ds(nk * 16, 16)] = row

            nk = nk + jnp.where(has, 1, 0).astype(jnp.int32)
            return (nk, rec_next) + tuple(new_scs)

        lax.fori_loop(0, K_MAX, body, (jnp.int32(0), rec0) + tuple(sc_regs))

        @pl.when(tid == 0)
        def _():
            pltpu.sync_copy(outv, out_hbm)


_sc_mesh = plsc.VectorSubcoreMesh(
    core_axis_name="c", subcore_axis_name="s", num_cores=2, num_subcores=16)

_sc_call = pl.kernel(
    _sc_nms,
    out_type=jax.ShapeDtypeStruct((K_MAX * 16,), jnp.float32),
    mesh=_sc_mesh,
    compiler_params=pltpu.CompilerParams(needs_layout_passes=False),
    scratch_types=[
        pltpu.VMEM((6 * PER_TILE,), jnp.float32),
        pltpu.VMEM((SLOT,), jnp.float32),
        pltpu.VMEM_SHARED((2, NTILES, SLOT), jnp.float32),
        pltpu.VMEM((NTILES, SLOT), jnp.float32),
        pltpu.VMEM((K_MAX * 16,), jnp.float32),
    ],
)


def _pad2d(v):
    return jnp.pad(v, (0, N_PAD - N_ANCHORS)).reshape(ROWS, COLS)


@jax.jit
def kernel(classification, regression, anchors):
    a = anchors[0]
    r = regression[0]
    parts = [_pad2d(a[:, k]) for k in range(4)]
    parts += [_pad2d(r[:, k]) for k in range(4)]
    parts.append(_pad2d(classification[0, :, 1]))
    planes = pl.pallas_call(
        _decode_body,
        out_shape=jax.ShapeDtypeStruct((6 * ROWS, COLS), jnp.float32),
        in_specs=[pl.BlockSpec((ROWS, COLS), lambda: (0, 0))] * 9,
        out_specs=pl.BlockSpec((6 * ROWS, COLS), lambda: (0, 0)),
    )(*parts)
    out = _sc_call(planes.reshape(6 * N_PAD))
    return out.reshape(K_MAX, 16)[:, :5]
